# baseline (speedup 1.0000x reference)
import functools

import jax
import jax.numpy as jnp
from jax import lax
from jax.experimental import pallas as pl
from jax.experimental.pallas import tpu as pltpu
from jax.experimental.pallas import tpu_sc as plsc

MAX_POS = 1024
D = 768
R = 4 * MAX_POS
L = 16
K = 1024
A = 128


@functools.lru_cache(maxsize=None)
def _make_kernel(N: int, NC: int, NS: int):
  NW = NC * NS
  CW = D // NW
  BLK = CW * A
  NB = N // A
  n_kpair = N // (2 * K)
  n_bpair = K // (2 * A)
  mesh = plsc.VectorSubcoreMesh(core_axis_name="c", subcore_axis_name="s",
                                num_cores=NC, num_subcores=NS)

  @functools.partial(
      pl.kernel,
      mesh=mesh,
      compiler_params=pltpu.CompilerParams(needs_layout_passes=False),
      out_type=jax.ShapeDtypeStruct((N * D,), jnp.float32),
      scratch_types=[
          pltpu.VMEM((CW * R,), jnp.float32),
          pltpu.VMEM((K,), jnp.int32), pltpu.VMEM((K,), jnp.int32),
          pltpu.VMEM((K,), jnp.int32), pltpu.VMEM((K,), jnp.int32),
          pltpu.VMEM((K,), jnp.int32), pltpu.VMEM((K,), jnp.int32),
          pltpu.VMEM((K,), jnp.int32), pltpu.VMEM((K,), jnp.int32),
          pltpu.VMEM((CW * A,), jnp.float32),
          pltpu.VMEM((CW * A,), jnp.float32),
          pltpu.SemaphoreType.DMA,
          pltpu.SemaphoreType.DMA,
          pltpu.SemaphoreType.DMA,
          pltpu.SemaphoreType.DMA,
          pltpu.SemaphoreType.DMA,
      ],
  )
  def k(tabs_flat, x1s, y1s, x2s, y2s, out,
        tab, bx0, bx1, by0, by1, bw0, bw1, bh0, bh1, acc0, acc1,
        tsem, isem0, isem1, osem0, osem1):
    wid = lax.axis_index("s") * NC + lax.axis_index("c")

    pltpu.async_copy(tabs_flat.at[pl.ds(wid * CW * R, CW * R)], tab, tsem)

    def fire_idx(kc, bufs, isem):
      base = kc * K
      pltpu.async_copy(x1s.at[pl.ds(base, K)], bufs[0], isem)
      pltpu.async_copy(y1s.at[pl.ds(base, K)], bufs[1], isem)
      pltpu.async_copy(x2s.at[pl.ds(base, K)], bufs[2], isem)
      pltpu.async_copy(y2s.at[pl.ds(base, K)], bufs[3], isem)

    def wait_idx(kc, bufs, isem):
      base = kc * K
      pltpu.make_async_copy(x1s.at[pl.ds(base, K)], bufs[0], isem).wait()
      pltpu.make_async_copy(y1s.at[pl.ds(base, K)], bufs[1], isem).wait()
      pltpu.make_async_copy(x2s.at[pl.ds(base, K)], bufs[2], isem).wait()
      pltpu.make_async_copy(y2s.at[pl.ds(base, K)], bufs[3], isem).wait()

    def out_desc(blk, acc, osem):
      return pltpu.make_async_copy(
          acc, out.at[pl.ds((wid * NB + blk) * BLK, BLK)], osem)

    ibufs0 = (bx0, by0, bw0, bh0)
    ibufs1 = (bx1, by1, bw1, bh1)

    fire_idx(0, ibufs0, isem0)
    pltpu.make_async_copy(
        tabs_flat.at[pl.ds(wid * CW * R, CW * R)], tab, tsem).wait()

    iota = lax.broadcasted_iota(jnp.int32, (L,), 0)

    def do_block(kc, b, bufs, acc, osem):
      blk = kc * (K // A) + b

      @pl.when(blk >= 2)
      def _():
        out_desc(blk - 2, acc, osem).wait()

      @plsc.parallel_loop(0, A // L)
      def _(g):
        sl = pl.ds(b * A + g * L, L)
        xv = bufs[0][sl]
        yv = bufs[1][sl] + MAX_POS
        wv = (bufs[2][sl] - xv) + 2 * MAX_POS
        hv = (bufs[3][sl] - yv) + 4 * MAX_POS
        tokv = iota + g * L
        for j in range(CW):
          s = (plsc.load_gather(tab, [xv + (j * R)])
               + plsc.load_gather(tab, [yv + (j * R)])
               + plsc.load_gather(tab, [wv + (j * R)])
               + plsc.load_gather(tab, [hv + (j * R)]))
          plsc.store_scatter(acc, [tokv + (j * A)], s)

      pltpu.async_copy(acc, out.at[pl.ds((wid * NB + blk) * BLK, BLK)], osem)

    def process_chunk(kc, bufs):
      def bpair(bb, carry):
        do_block(kc, 2 * bb, bufs, acc0, osem0)
        do_block(kc, 2 * bb + 1, bufs, acc1, osem1)
        return carry
      lax.fori_loop(0, n_bpair, bpair, 0)

    def kpair(kk, carry):
      ke = 2 * kk
      fire_idx(ke + 1, ibufs1, isem1)
      wait_idx(ke, ibufs0, isem0)
      process_chunk(ke, ibufs0)

      @pl.when(kk + 1 < n_kpair)
      def _():
        fire_idx(ke + 2, ibufs0, isem0)
      wait_idx(ke + 1, ibufs1, isem1)
      process_chunk(ke + 1, ibufs1)
      return carry

    lax.fori_loop(0, n_kpair, kpair, 0)
    out_desc(NB - 2, acc0, osem0).wait()
    out_desc(NB - 1, acc1, osem1).wait()

  return k


def kernel(bboxes, x_table, y_table, h_table, w_table):
  B, S, _ = bboxes.shape
  N = B * S
  bb = bboxes.reshape(N, 4)
  tabs_flat = jnp.concatenate(
      [x_table, y_table, w_table, h_table], axis=0).T.reshape(-1)
  info = plsc.get_sparse_core_info()
  NW = info.num_cores * info.num_subcores
  CW = D // NW
  k = _make_kernel(N, info.num_cores, info.num_subcores)
  flat = k(tabs_flat, bb[:, 0], bb[:, 1], bb[:, 2], bb[:, 3])
  out = flat.reshape(NW, N // A, CW, A).transpose(1, 3, 0, 2).reshape(N, D)
  return out.reshape(B, S, D)


# speedup vs baseline: 2.2054x; 2.2054x over previous
import functools

import jax
import jax.numpy as jnp
from jax import lax
from jax.experimental import pallas as pl
from jax.experimental.pallas import tpu as pltpu
from jax.experimental.pallas import tpu_sc as plsc

MAX_POS = 1024
D = 768
L = 16
C = 16


@functools.lru_cache(maxsize=None)
def _make_kernel(N: int, NC: int, NS: int):
  NW = NC * NS
  assert N % NW == 0
  b_per_w = N // NW
  assert b_per_w % (2 * C) == 0
  n_half = b_per_w // (2 * C)
  mesh = plsc.VectorSubcoreMesh(core_axis_name="c", subcore_axis_name="s",
                                num_cores=NC, num_subcores=NS)

  @functools.partial(
      pl.kernel,
      mesh=mesh,
      out_type=jax.ShapeDtypeStruct((N, D), jnp.float32),
      scratch_types=[
          pltpu.VMEM((b_per_w,), jnp.int32),
          pltpu.VMEM((b_per_w,), jnp.int32),
          pltpu.VMEM((b_per_w,), jnp.int32),
          pltpu.VMEM((b_per_w,), jnp.int32),
          pltpu.VMEM((4 * C,), jnp.int32),
          pltpu.VMEM((4 * C,), jnp.int32),
          pltpu.VMEM((4 * C, D), jnp.float32),
          pltpu.VMEM((4 * C, D), jnp.float32),
          pltpu.VMEM((C, D), jnp.float32),
          pltpu.VMEM((C, D), jnp.float32),
          pltpu.SemaphoreType.DMA,
          pltpu.SemaphoreType.DMA,
          pltpu.SemaphoreType.DMA,
          pltpu.SemaphoreType.DMA,
      ],
  )
  def k(tables, x1s, y1s, x2s, y2s, out,
        ix1, iy1, ix2, iy2, idx0, idx1,
        rows0, rows1, acc0, acc1, sem0, sem1, osem0, osem1):
    wid = lax.axis_index("s") * NC + lax.axis_index("c")
    wbase = wid * b_per_w

    pltpu.sync_copy(x1s.at[pl.ds(wbase, b_per_w)], ix1)
    pltpu.sync_copy(y1s.at[pl.ds(wbase, b_per_w)], iy1)
    pltpu.sync_copy(x2s.at[pl.ds(wbase, b_per_w)], ix2)
    pltpu.sync_copy(y2s.at[pl.ds(wbase, b_per_w)], iy2)

    def build_idx(g, idx):
      cbase = g * C
      for v in range(C // L):
        src = pl.ds(cbase + v * L, L)
        a = ix1[src]
        b = iy1[src]
        idx[pl.ds(v * L, L)] = a
        idx[pl.ds(C + v * L, L)] = b + MAX_POS
        idx[pl.ds(2 * C + v * L, L)] = (ix2[src] - a) + 2 * MAX_POS
        idx[pl.ds(3 * C + v * L, L)] = (iy2[src] - b) + 3 * MAX_POS

    def do_sum(rows, acc):
      @plsc.parallel_loop(0, C)
      def _(c):
        for j in range(D // L):
          sl = pl.ds(j * L, L)
          acc[c, sl] = (rows[c, sl] + rows[C + c, sl]
                        + rows[2 * C + c, sl] + rows[3 * C + c, sl])

    def out_desc(g, acc, osem):
      return pltpu.make_async_copy(acc, out.at[pl.ds(wbase + g * C, C)], osem)

    build_idx(0, idx0)
    pltpu.async_copy(tables.at[idx0], rows0, sem0)

    def half_body(t, carry):
      ge = 2 * t
      build_idx(ge + 1, idx1)
      pltpu.async_copy(tables.at[idx1], rows1, sem1)
      pltpu.make_async_copy(tables.at[idx0], rows0, sem0).wait()

      @pl.when(t > 0)
      def _():
        out_desc(ge - 2, acc0, osem0).wait()
      do_sum(rows0, acc0)
      pltpu.async_copy(acc0, out.at[pl.ds(wbase + ge * C, C)], osem0)

      @pl.when(t + 1 < n_half)
      def _():
        build_idx(ge + 2, idx0)
        pltpu.async_copy(tables.at[idx0], rows0, sem0)
      pltpu.make_async_copy(tables.at[idx1], rows1, sem1).wait()

      @pl.when(t > 0)
      def _():
        out_desc(ge - 1, acc1, osem1).wait()
      do_sum(rows1, acc1)
      pltpu.async_copy(acc1, out.at[pl.ds(wbase + (ge + 1) * C, C)], osem1)
      return carry

    lax.fori_loop(0, n_half, half_body, 0)
    out_desc(2 * n_half - 2, acc0, osem0).wait()
    out_desc(2 * n_half - 1, acc1, osem1).wait()

  return k


BM = 512
M_TC = 8192


@functools.lru_cache(maxsize=None)
def _make_tc_kernel(M: int):
  grid = M // BM

  def body(x1r, y1r, x2r, y2r, xt, yt, wt, ht, o):
    x1v = x1r[0]
    y1v = y1r[0]
    wv = x2r[0] - x1v
    hv = y2r[0] - y1v
    iota = lax.broadcasted_iota(jnp.int32, (BM, MAX_POS), 1)

    def emb(idxv, tbl):
      oh = jnp.where(idxv == iota, 1.0, 0.0).astype(jnp.bfloat16)
      return lax.dot_general(oh, tbl[...], (((1,), (0,)), ((), ())),
                             preferred_element_type=jnp.float32)

    o[...] = (emb(x1v, xt) + emb(y1v, yt) + emb(wv, wt) + emb(hv, ht))

  iidx = lambda i: (i, 0, 0)
  tidx = lambda i: (0, 0)
  return pl.pallas_call(
      body,
      grid=(grid,),
      in_specs=[
          pl.BlockSpec((1, BM, 1), iidx),
          pl.BlockSpec((1, BM, 1), iidx),
          pl.BlockSpec((1, BM, 1), iidx),
          pl.BlockSpec((1, BM, 1), iidx),
          pl.BlockSpec((MAX_POS, D), tidx),
          pl.BlockSpec((MAX_POS, D), tidx),
          pl.BlockSpec((MAX_POS, D), tidx),
          pl.BlockSpec((MAX_POS, D), tidx),
      ],
      out_specs=pl.BlockSpec((BM, D), lambda i: (i, 0)),
      out_shape=jax.ShapeDtypeStruct((M, D), jnp.float32),
  )


def kernel(bboxes, x_table, y_table, h_table, w_table):
  B, S, _ = bboxes.shape
  N = B * S
  bb = bboxes.reshape(N, 4)
  tables = jnp.concatenate([x_table, y_table, w_table, h_table], axis=0)
  info = plsc.get_sparse_core_info()
  M = M_TC
  NSC = N - M

  k_sc = _make_kernel(NSC, info.num_cores, info.num_subcores)
  out_sc = k_sc(tables, bb[M:, 0], bb[M:, 1], bb[M:, 2], bb[M:, 3])

  k_tc = _make_tc_kernel(M)
  nb = M // BM
  r3 = lambda a: a.reshape(nb, BM, 1)
  out_tc = k_tc(r3(bb[:M, 0]), r3(bb[:M, 1]), r3(bb[:M, 2]), r3(bb[:M, 3]),
                x_table.astype(jnp.bfloat16), y_table.astype(jnp.bfloat16),
                w_table.astype(jnp.bfloat16), h_table.astype(jnp.bfloat16))

  out = jnp.concatenate([out_tc, out_sc], axis=0)
  return out.reshape(B, S, D)


# speedup vs baseline: 2.4441x; 1.1082x over previous
import functools

import jax
import jax.numpy as jnp
from jax import lax
from jax.experimental import pallas as pl
from jax.experimental.pallas import tpu as pltpu
from jax.experimental.pallas import tpu_sc as plsc

MAX_POS = 1024
D = 768
L = 16
C = 16


@functools.lru_cache(maxsize=None)
def _make_kernel(N: int, NC: int, NS: int):
  NW = NC * NS
  assert N % NW == 0
  b_per_w = N // NW
  assert b_per_w % (2 * C) == 0
  n_half = b_per_w // (2 * C)
  mesh = plsc.VectorSubcoreMesh(core_axis_name="c", subcore_axis_name="s",
                                num_cores=NC, num_subcores=NS)

  @functools.partial(
      pl.kernel,
      mesh=mesh,
      out_type=jax.ShapeDtypeStruct((N, D), jnp.float32),
      scratch_types=[
          pltpu.VMEM((b_per_w,), jnp.int32),
          pltpu.VMEM((b_per_w,), jnp.int32),
          pltpu.VMEM((b_per_w,), jnp.int32),
          pltpu.VMEM((b_per_w,), jnp.int32),
          pltpu.VMEM((4 * C,), jnp.int32),
          pltpu.VMEM((4 * C,), jnp.int32),
          pltpu.VMEM((4 * C, D), jnp.float32),
          pltpu.VMEM((4 * C, D), jnp.float32),
          pltpu.VMEM((C, D), jnp.float32),
          pltpu.VMEM((C, D), jnp.float32),
          pltpu.SemaphoreType.DMA,
          pltpu.SemaphoreType.DMA,
          pltpu.SemaphoreType.DMA,
          pltpu.SemaphoreType.DMA,
      ],
  )
  def k(tables, x1s, y1s, x2s, y2s, out,
        ix1, iy1, ix2, iy2, idx0, idx1,
        rows0, rows1, acc0, acc1, sem0, sem1, osem0, osem1):
    wid = lax.axis_index("s") * NC + lax.axis_index("c")
    wbase = wid * b_per_w

    pltpu.sync_copy(x1s.at[pl.ds(wbase, b_per_w)], ix1)
    pltpu.sync_copy(y1s.at[pl.ds(wbase, b_per_w)], iy1)
    pltpu.sync_copy(x2s.at[pl.ds(wbase, b_per_w)], ix2)
    pltpu.sync_copy(y2s.at[pl.ds(wbase, b_per_w)], iy2)

    def build_idx(g, idx):
      cbase = g * C
      for v in range(C // L):
        src = pl.ds(cbase + v * L, L)
        a = ix1[src]
        b = iy1[src]
        idx[pl.ds(v * L, L)] = a
        idx[pl.ds(C + v * L, L)] = b + MAX_POS
        idx[pl.ds(2 * C + v * L, L)] = (ix2[src] - a) + 2 * MAX_POS
        idx[pl.ds(3 * C + v * L, L)] = (iy2[src] - b) + 3 * MAX_POS

    def do_sum(rows, acc):
      @plsc.parallel_loop(0, C)
      def _(c):
        for j in range(D // L):
          sl = pl.ds(j * L, L)
          acc[c, sl] = (rows[c, sl] + rows[C + c, sl]
                        + rows[2 * C + c, sl] + rows[3 * C + c, sl])

    def out_desc(g, acc, osem):
      return pltpu.make_async_copy(acc, out.at[pl.ds(wbase + g * C, C)], osem)

    build_idx(0, idx0)
    pltpu.async_copy(tables.at[idx0], rows0, sem0)

    def half_body(t, carry):
      ge = 2 * t
      build_idx(ge + 1, idx1)
      pltpu.async_copy(tables.at[idx1], rows1, sem1)
      pltpu.make_async_copy(tables.at[idx0], rows0, sem0).wait()

      @pl.when(t > 0)
      def _():
        out_desc(ge - 2, acc0, osem0).wait()
      do_sum(rows0, acc0)
      pltpu.async_copy(acc0, out.at[pl.ds(wbase + ge * C, C)], osem0)

      @pl.when(t + 1 < n_half)
      def _():
        build_idx(ge + 2, idx0)
        pltpu.async_copy(tables.at[idx0], rows0, sem0)
      pltpu.make_async_copy(tables.at[idx1], rows1, sem1).wait()

      @pl.when(t > 0)
      def _():
        out_desc(ge - 1, acc1, osem1).wait()
      do_sum(rows1, acc1)
      pltpu.async_copy(acc1, out.at[pl.ds(wbase + (ge + 1) * C, C)], osem1)
      return carry

    lax.fori_loop(0, n_half, half_body, 0)
    out_desc(2 * n_half - 2, acc0, osem0).wait()
    out_desc(2 * n_half - 1, acc1, osem1).wait()

  return k


def kernel(bboxes, x_table, y_table, h_table, w_table):
  B, S, _ = bboxes.shape
  N = B * S
  bb = bboxes.reshape(N, 4)
  tables = jnp.concatenate([x_table, y_table, w_table, h_table], axis=0)
  info = plsc.get_sparse_core_info()
  k = _make_kernel(N, info.num_cores, info.num_subcores)
  out = k(tables, bb[:, 0], bb[:, 1], bb[:, 2], bb[:, 3])
  return out.reshape(B, S, D)


# speedup vs baseline: 3.4149x; 1.3972x over previous
import functools

import jax
import jax.numpy as jnp
from jax import lax
from jax.experimental import pallas as pl
from jax.experimental.pallas import tpu as pltpu
from jax.experimental.pallas import tpu_sc as plsc

MAX_POS = 1024
D = 768
L = 16
C = 16


@functools.lru_cache(maxsize=None)
def _make_kernel(N: int, NC: int, NS: int):
  NW = NC * NS
  assert N % NW == 0
  b_per_w = N // NW
  assert b_per_w % (2 * C) == 0
  n_half = b_per_w // (2 * C)
  mesh = plsc.VectorSubcoreMesh(core_axis_name="c", subcore_axis_name="s",
                                num_cores=NC, num_subcores=NS)

  @functools.partial(
      pl.kernel,
      mesh=mesh,
      compiler_params=pltpu.CompilerParams(needs_layout_passes=False),
      out_type=jax.ShapeDtypeStruct((N, D), jnp.float32),
      scratch_types=[
          pltpu.VMEM((b_per_w,), jnp.int32),
          pltpu.VMEM((b_per_w,), jnp.int32),
          pltpu.VMEM((b_per_w,), jnp.int32),
          pltpu.VMEM((b_per_w,), jnp.int32),
          pltpu.VMEM((4 * C,), jnp.int32),
          pltpu.VMEM((4 * C,), jnp.int32),
          pltpu.VMEM((4 * C, D // 2), jnp.int32),
          pltpu.VMEM((4 * C, D // 2), jnp.int32),
          pltpu.VMEM((C, D), jnp.float32),
          pltpu.VMEM((C, D), jnp.float32),
          pltpu.SemaphoreType.DMA,
          pltpu.SemaphoreType.DMA,
          pltpu.SemaphoreType.DMA,
          pltpu.SemaphoreType.DMA,
      ],
  )
  def k(tables, x1s, y1s, x2s, y2s, out,
        ix1, iy1, ix2, iy2, idx0, idx1,
        rows0, rows1, acc0, acc1, sem0, sem1, osem0, osem1):
    wid = lax.axis_index("s") * NC + lax.axis_index("c")
    wbase = wid * b_per_w

    pltpu.sync_copy(x1s.at[pl.ds(wbase, b_per_w)], ix1)
    pltpu.sync_copy(y1s.at[pl.ds(wbase, b_per_w)], iy1)
    pltpu.sync_copy(x2s.at[pl.ds(wbase, b_per_w)], ix2)
    pltpu.sync_copy(y2s.at[pl.ds(wbase, b_per_w)], iy2)

    def build_idx(g, idx):
      cbase = g * C
      for v in range(C // L):
        src = pl.ds(cbase + v * L, L)
        a = ix1[src]
        b = iy1[src]
        idx[pl.ds(v * L, L)] = a
        idx[pl.ds(C + v * L, L)] = b + MAX_POS
        idx[pl.ds(2 * C + v * L, L)] = (ix2[src] - a) + 2 * MAX_POS
        idx[pl.ds(3 * C + v * L, L)] = (iy2[src] - b) + 3 * MAX_POS

    def do_sum(rows, acc):
      hi_mask = jnp.int32(-65536)

      def halves(w):
        lo = plsc.bitcast(w << 16, jnp.float32)
        hi = plsc.bitcast(w & hi_mask, jnp.float32)
        return lo, hi

      @plsc.parallel_loop(0, C)
      def _(c):
        for j in range(D // (2 * L)):
          sl = pl.ds(j * L, L)
          a0, b0 = halves(rows[c, sl])
          a1, b1 = halves(rows[C + c, sl])
          a2, b2 = halves(rows[2 * C + c, sl])
          a3, b3 = halves(rows[3 * C + c, sl])
          acc[c, pl.ds(j * 2 * L, L)] = (a0 + a1) + (a2 + a3)
          acc[c, pl.ds(j * 2 * L + L, L)] = (b0 + b1) + (b2 + b3)

    def out_desc(g, acc, osem):
      return pltpu.make_async_copy(acc, out.at[pl.ds(wbase + g * C, C)], osem)

    build_idx(0, idx0)
    pltpu.async_copy(tables.at[idx0], rows0, sem0)

    def half_body(t, carry):
      ge = 2 * t
      build_idx(ge + 1, idx1)
      pltpu.async_copy(tables.at[idx1], rows1, sem1)
      pltpu.make_async_copy(tables.at[idx0], rows0, sem0).wait()

      @pl.when(t > 0)
      def _():
        out_desc(ge - 2, acc0, osem0).wait()
      do_sum(rows0, acc0)
      pltpu.async_copy(acc0, out.at[pl.ds(wbase + ge * C, C)], osem0)

      @pl.when(t + 1 < n_half)
      def _():
        build_idx(ge + 2, idx0)
        pltpu.async_copy(tables.at[idx0], rows0, sem0)
      pltpu.make_async_copy(tables.at[idx1], rows1, sem1).wait()

      @pl.when(t > 0)
      def _():
        out_desc(ge - 1, acc1, osem1).wait()
      do_sum(rows1, acc1)
      pltpu.async_copy(acc1, out.at[pl.ds(wbase + (ge + 1) * C, C)], osem1)
      return carry

    lax.fori_loop(0, n_half, half_body, 0)
    out_desc(2 * n_half - 2, acc0, osem0).wait()
    out_desc(2 * n_half - 1, acc1, osem1).wait()

  return k


def kernel(bboxes, x_table, y_table, h_table, w_table):
  B, S, _ = bboxes.shape
  N = B * S
  bb = bboxes.reshape(N, 4)
  tables = jnp.concatenate([x_table, y_table, w_table, h_table], axis=0)
  tables = (tables.astype(jnp.bfloat16)
            .reshape(4 * MAX_POS, D // 32, 2, 16)
            .transpose(0, 1, 3, 2)
            .reshape(4 * MAX_POS, D // 2, 2))
  tables = lax.bitcast_convert_type(tables, jnp.int32)
  info = plsc.get_sparse_core_info()
  k = _make_kernel(N, info.num_cores, info.num_subcores)
  out = k(tables, bb[:, 0], bb[:, 1], bb[:, 2], bb[:, 3])
  return out.reshape(B, S, D)
